# baseline (speedup 1.0000x reference)
import jax
import jax.numpy as jnp
import numpy as np
from jax.experimental import pallas as pl
from jax.experimental.pallas import tpu as pltpu

D = 1024
H = 16
DN = 32
DR = 32
DV = 64
L = 256
E = 8
TOPK = 2
F = 512
THETA = 10000.0
ALPHA = 0.01
EPS = 1e-6
SCALE = 1.0 / np.sqrt(DN + DR)

SB = 512
QB = 256
KB = 256
DV2 = 128

_F32 = jnp.float32
_BF16 = jnp.bfloat16


def _dot(a, b):
    return jax.lax.dot_general(a, b, (((1,), (0,)), ((), ())),
                               preferred_element_type=_F32)


def _dot_t(a, b):
    return jax.lax.dot_general(a, b, (((1,), (1,)), ((), ())),
                               preferred_element_type=_F32)


def _prep_kernel(x_ref, cos_ref, sin_ref, cosq_ref, sinq_ref, n1_ref,
                 wqn_ref, wq1_ref, wq2_ref, wdl_ref, wd1_ref, wd2_ref,
                 kvn_ref, wkn_ref, wv_ref,
                 qc_out, kc_out, v_out):
    sb = x_ref.shape[0]
    x = x_ref[...]
    h = x * jax.lax.rsqrt(jnp.mean(x * x, axis=1, keepdims=True) + EPS)
    h = h * n1_ref[...]
    hb = h.astype(_BF16)

    qn = _dot(hb, wqn_ref[...])
    q1 = _dot(hb, wq1_ref[...])
    q2 = _dot(hb, wq2_ref[...])
    cq = cosq_ref[...]
    sq = sinq_ref[...]
    rq1 = q1 * cq - q2 * sq
    rq2 = q1 * sq + q2 * cq
    qc = jnp.concatenate(
        [(qn * SCALE).reshape(sb, H, DN),
         (rq1 * SCALE).reshape(sb, H, DR // 2),
         (rq2 * SCALE).reshape(sb, H, DR // 2)], axis=2)
    qc_out[...] = qc.reshape(sb, H * (DN + DR)).astype(_BF16)

    c = _dot(hb, wdl_ref[...])
    p1 = _dot(hb, wd1_ref[...])
    p2 = _dot(hb, wd2_ref[...])
    co = cos_ref[...]
    si = sin_ref[...]
    rk1 = p1 * co - p2 * si
    rk2 = p1 * si + p2 * co

    cn = c * jax.lax.rsqrt(jnp.mean(c * c, axis=1, keepdims=True) + EPS)
    cn = cn * kvn_ref[...]
    cb = cn.astype(_BF16)
    kn = _dot(cb, wkn_ref[...])
    kc = jnp.concatenate(
        [kn.reshape(sb, H, DN),
         jnp.broadcast_to(rk1[:, None, :], (sb, H, DR // 2)),
         jnp.broadcast_to(rk2[:, None, :], (sb, H, DR // 2))], axis=2)
    kc_out[...] = kc.reshape(sb, H * (DN + DR)).astype(_BF16)
    vv = _dot(cb, wv_ref[...]).reshape(sb, H, DV)
    vext = jnp.concatenate(
        [vv, jnp.ones((sb, H, 1), _F32), jnp.zeros((sb, H, DV2 - DV - 1), _F32)],
        axis=2)
    v_out[...] = vext.reshape(sb, H * DV2).astype(_BF16)


def _attn_kernel(qc_ref, kc_ref, v_ref, o_ref, s_scr, mx_scr):
    iq = pl.program_id(1)
    nb = iq + 1
    qc = qc_ref[0]
    qiota = jax.lax.broadcasted_iota(jnp.int32, (QB, KB), 0)
    kiota = jax.lax.broadcasted_iota(jnp.int32, (QB, KB), 1)

    def qk(j, _):
        s = _dot_t(qc, kc_ref[0, pl.ds(j * KB, KB), :])
        s = jnp.where(j * KB + kiota <= iq * QB + qiota, s, -1e30)
        s_scr[j] = s
        mx_scr[j] = jnp.max(s, axis=1, keepdims=True)
        return 0

    jax.lax.fori_loop(0, nb, qk, 0, unroll=False)

    def mx(j, m):
        return jnp.maximum(m, mx_scr[j])

    m = jax.lax.fori_loop(0, nb, mx, jnp.full((QB, 1), -1e30, _F32),
                          unroll=False)

    def pv(j, acc):
        p = jnp.exp(s_scr[j] - m).astype(_BF16)
        return acc + _dot(p, v_ref[0, pl.ds(j * KB, KB), :])

    acc = jax.lax.fori_loop(0, nb, pv, jnp.zeros((QB, DV2), _F32),
                            unroll=False)
    o_ref[0] = (acc[:, :DV] / acc[:, DV:DV + 1]).astype(_BF16)


def _proj_kernel(x_ref, a_ref, wo_ref, n2_ref, rw_ref, x1_ref, h2_ref,
                 lg_ref):
    x1 = x_ref[...] + _dot(a_ref[...], wo_ref[...])
    x1_ref[...] = x1
    h2 = x1 * jax.lax.rsqrt(jnp.mean(x1 * x1, axis=1, keepdims=True) + EPS)
    h2 = h2 * n2_ref[...]
    h2_ref[...] = h2.astype(_BF16)
    lg_ref[...] = _dot(h2, rw_ref[...])


def _router_kernel(lg_ref, gates_ref, aux_ref):
    lg = lg_ref[...]
    T = lg.shape[0]
    mx = jnp.max(lg, axis=1, keepdims=True)
    ex = jnp.exp(lg - mx)
    p = ex / jnp.sum(ex, axis=1, keepdims=True)
    iota = jax.lax.broadcasted_iota(jnp.int32, p.shape, 1)
    m1 = jnp.max(p, axis=1, keepdims=True)
    i1 = jnp.min(jnp.where(p >= m1, iota, E), axis=1, keepdims=True)
    oh1 = iota == i1
    pm = jnp.where(oh1, -1.0, p)
    m2 = jnp.max(pm, axis=1, keepdims=True)
    i2 = jnp.min(jnp.where(pm >= m2, iota, E), axis=1, keepdims=True)
    oh2 = iota == i2
    den = m1 + m2
    gates_ref[...] = (jnp.where(oh1, m1 / den, 0.0)
                      + jnp.where(oh2, m2 / den, 0.0))
    cnt = jnp.sum(oh1.astype(_F32) + oh2.astype(_F32), axis=0, keepdims=True)
    pmean = jnp.sum(p, axis=0, keepdims=True)
    f = cnt / (T * TOPK)
    pm_avg = pmean / T
    aux_ref[...] = ALPHA * E * jnp.sum(f * pm_avg, axis=1, keepdims=True)


def _moe_kernel(h2_ref, x1_ref, gates_ref, w1_ref, b1_ref, w2_ref, b2_ref,
                out_ref):
    e = pl.program_id(0)

    @pl.when(e == 0)
    def _():
        out_ref[...] = x1_ref[...]

    up = _dot(h2_ref[...], w1_ref[0]) + b1_ref[0]
    act = jax.nn.gelu(up)
    dn = _dot(act.astype(_BF16), w2_ref[0]) + b2_ref[0]
    gates = gates_ref[...]
    lane = jax.lax.broadcasted_iota(jnp.int32, gates.shape, 1)
    g = jnp.sum(jnp.where(lane == e, gates, 0.0), axis=1, keepdims=True)
    out_ref[...] += dn * g


def kernel(x, position_ids, norm1_w, Wq, Wdkv, kv_norm_w, Wukv, Wo, norm2_w,
           router_w, W1, b1, W2, b2):
    B, S, _ = x.shape
    x2d = x.reshape(S, D)

    inv = 1.0 / (THETA ** (jnp.arange(0, DR, 2, dtype=_F32) / DR))
    ang = position_ids.reshape(S).astype(_F32)[:, None] * inv
    cos = jnp.cos(ang)
    sin = jnp.sin(ang)
    cosq = jnp.tile(cos, (1, H))
    sinq = jnp.tile(sin, (1, H))

    Wq3 = Wq.reshape(D, H, DN + DR)
    wqn = Wq3[:, :, :DN].reshape(D, H * DN).astype(_BF16)
    wq1 = Wq3[:, :, DN::2].reshape(D, H * (DR // 2)).astype(_BF16)
    wq2 = Wq3[:, :, DN + 1::2].reshape(D, H * (DR // 2)).astype(_BF16)
    wdl = Wdkv[:, :L].astype(_BF16)
    wd1 = Wdkv[:, L::2].astype(_BF16)
    wd2 = Wdkv[:, L + 1::2].astype(_BF16)
    Wukv3 = Wukv.reshape(L, H, DN + DV)
    wkn = Wukv3[:, :, :DN].reshape(L, H * DN).astype(_BF16)
    wv = Wukv3[:, :, DN:].reshape(L, H * DV).astype(_BF16)
    wo = Wo.astype(_BF16)
    w1 = W1.astype(_BF16)
    w2 = W2.astype(_BF16)
    n1 = norm1_w.reshape(1, D)
    n2 = norm2_w.reshape(1, D)
    kvn = kv_norm_w.reshape(1, L)

    HR = H * (DR // 2)

    nsb = S // SB
    qc, kc, v = pl.pallas_call(
        _prep_kernel,
        grid=(nsb,),
        in_specs=[
            pl.BlockSpec((SB, D), lambda i: (i, 0)),
            pl.BlockSpec((SB, DR // 2), lambda i: (i, 0)),
            pl.BlockSpec((SB, DR // 2), lambda i: (i, 0)),
            pl.BlockSpec((SB, HR), lambda i: (i, 0)),
            pl.BlockSpec((SB, HR), lambda i: (i, 0)),
            pl.BlockSpec((1, D), lambda i: (0, 0)),
            pl.BlockSpec((D, H * DN), lambda i: (0, 0)),
            pl.BlockSpec((D, HR), lambda i: (0, 0)),
            pl.BlockSpec((D, HR), lambda i: (0, 0)),
            pl.BlockSpec((D, L), lambda i: (0, 0)),
            pl.BlockSpec((D, DR // 2), lambda i: (0, 0)),
            pl.BlockSpec((D, DR // 2), lambda i: (0, 0)),
            pl.BlockSpec((1, L), lambda i: (0, 0)),
            pl.BlockSpec((L, H * DN), lambda i: (0, 0)),
            pl.BlockSpec((L, H * DV), lambda i: (0, 0)),
        ],
        out_specs=[
            pl.BlockSpec((SB, H * (DN + DR)), lambda i: (i, 0)),
            pl.BlockSpec((SB, H * (DN + DR)), lambda i: (i, 0)),
            pl.BlockSpec((SB, H * DV2), lambda i: (i, 0)),
        ],
        out_shape=[
            jax.ShapeDtypeStruct((S, H * (DN + DR)), _BF16),
            jax.ShapeDtypeStruct((S, H * (DN + DR)), _BF16),
            jax.ShapeDtypeStruct((S, H * DV2), _BF16),
        ],
    )(x2d, cos, sin, cosq, sinq, n1, wqn, wq1, wq2, wdl, wd1, wd2, kvn,
      wkn, wv)

    DH = DN + DR
    qc3 = qc.reshape(S, H, DH).transpose(1, 0, 2)
    kc3 = kc.reshape(S, H, DH).transpose(1, 0, 2)
    v3 = v.reshape(S, H, DV2).transpose(1, 0, 2)

    nq = S // QB
    attn3 = pl.pallas_call(
        _attn_kernel,
        grid=(H, nq),
        in_specs=[
            pl.BlockSpec((1, QB, DH), lambda h, i: (h, i, 0)),
            pl.BlockSpec((1, S, DH), lambda h, i: (h, 0, 0)),
            pl.BlockSpec((1, S, DV2), lambda h, i: (h, 0, 0)),
        ],
        out_specs=pl.BlockSpec((1, QB, DV), lambda h, i: (h, i, 0)),
        out_shape=jax.ShapeDtypeStruct((H, S, DV), _BF16),
        scratch_shapes=[
            pltpu.VMEM((S // KB, QB, KB), _F32),
            pltpu.VMEM((S // KB, QB, 1), _F32),
        ],
    )(qc3, kc3, v3)
    attn = attn3.transpose(1, 0, 2).reshape(S, H * DV)

    x1, h2, logits = pl.pallas_call(
        _proj_kernel,
        grid=(nsb,),
        in_specs=[
            pl.BlockSpec((SB, D), lambda i: (i, 0)),
            pl.BlockSpec((SB, H * DV), lambda i: (i, 0)),
            pl.BlockSpec((H * DV, D), lambda i: (0, 0)),
            pl.BlockSpec((1, D), lambda i: (0, 0)),
            pl.BlockSpec((D, E), lambda i: (0, 0)),
        ],
        out_specs=[
            pl.BlockSpec((SB, D), lambda i: (i, 0)),
            pl.BlockSpec((SB, D), lambda i: (i, 0)),
            pl.BlockSpec((SB, E), lambda i: (i, 0)),
        ],
        out_shape=[
            jax.ShapeDtypeStruct((S, D), _F32),
            jax.ShapeDtypeStruct((S, D), _BF16),
            jax.ShapeDtypeStruct((S, E), _F32),
        ],
    )(x2d, attn, wo, n2, router_w)

    gates, aux = pl.pallas_call(
        _router_kernel,
        grid=(1,),
        in_specs=[pl.BlockSpec((S, E), lambda i: (0, 0))],
        out_specs=[
            pl.BlockSpec((S, E), lambda i: (0, 0)),
            pl.BlockSpec((1, 1), lambda i: (0, 0)),
        ],
        out_shape=[
            jax.ShapeDtypeStruct((S, E), _F32),
            jax.ShapeDtypeStruct((1, 1), _F32),
        ],
    )(logits)

    out = pl.pallas_call(
        _moe_kernel,
        grid=(E,),
        in_specs=[
            pl.BlockSpec((S, D), lambda e: (0, 0)),
            pl.BlockSpec((S, D), lambda e: (0, 0)),
            pl.BlockSpec((S, E), lambda e: (0, 0)),
            pl.BlockSpec((1, D, F), lambda e: (e, 0, 0)),
            pl.BlockSpec((1, 1, F), lambda e: (e, 0, 0)),
            pl.BlockSpec((1, F, D), lambda e: (e, 0, 0)),
            pl.BlockSpec((1, 1, D), lambda e: (e, 0, 0)),
        ],
        out_specs=pl.BlockSpec((S, D), lambda e: (0, 0)),
        out_shape=jax.ShapeDtypeStruct((S, D), _F32),
    )(h2, x1, gates, w1, b1.reshape(E, 1, F), w2, b2.reshape(E, 1, D))

    return (out.reshape(B, S, D), aux[0, 0])


# speedup vs baseline: 1.2283x; 1.2283x over previous
import jax
import jax.numpy as jnp
import numpy as np
from jax.experimental import pallas as pl
from jax.experimental.pallas import tpu as pltpu

D = 1024
H = 16
DN = 32
DR = 32
DV = 64
L = 256
E = 8
TOPK = 2
F = 512
THETA = 10000.0
ALPHA = 0.01
EPS = 1e-6
SCALE = 1.0 / np.sqrt(DN + DR)

SB = 512
QB = 256
KB = 256
DV2 = 128
HB = 2

_F32 = jnp.float32
_BF16 = jnp.bfloat16


def _dot(a, b):
    return jax.lax.dot_general(a, b, (((1,), (0,)), ((), ())),
                               preferred_element_type=_F32)


def _dot_t(a, b):
    return jax.lax.dot_general(a, b, (((1,), (1,)), ((), ())),
                               preferred_element_type=_F32)


def _prep_kernel(x_ref, cos_ref, sin_ref, cosq_ref, sinq_ref, n1_ref,
                 wqn_ref, wq1_ref, wq2_ref, wdl_ref, wd1_ref, wd2_ref,
                 kvn_ref, wkn_ref, wv_ref,
                 qc_out, kc_out, v_out):
    sb = x_ref.shape[0]
    x = x_ref[...]
    h = x * jax.lax.rsqrt(jnp.mean(x * x, axis=1, keepdims=True) + EPS)
    h = h * n1_ref[...]
    hb = h.astype(_BF16)

    qn = _dot(hb, wqn_ref[...])
    q1 = _dot(hb, wq1_ref[...])
    q2 = _dot(hb, wq2_ref[...])
    cq = cosq_ref[...]
    sq = sinq_ref[...]
    rq1 = q1 * cq - q2 * sq
    rq2 = q1 * sq + q2 * cq
    qc = jnp.concatenate(
        [(qn * SCALE).reshape(sb, H, DN),
         (rq1 * SCALE).reshape(sb, H, DR // 2),
         (rq2 * SCALE).reshape(sb, H, DR // 2)], axis=2)
    qc_out[...] = qc.reshape(sb, H * (DN + DR)).astype(_BF16)

    c = _dot(hb, wdl_ref[...])
    p1 = _dot(hb, wd1_ref[...])
    p2 = _dot(hb, wd2_ref[...])
    co = cos_ref[...]
    si = sin_ref[...]
    rk1 = p1 * co - p2 * si
    rk2 = p1 * si + p2 * co

    cn = c * jax.lax.rsqrt(jnp.mean(c * c, axis=1, keepdims=True) + EPS)
    cn = cn * kvn_ref[...]
    cb = cn.astype(_BF16)
    kn = _dot(cb, wkn_ref[...])
    kc = jnp.concatenate(
        [kn.reshape(sb, H, DN),
         jnp.broadcast_to(rk1[:, None, :], (sb, H, DR // 2)),
         jnp.broadcast_to(rk2[:, None, :], (sb, H, DR // 2))], axis=2)
    kc_out[...] = kc.reshape(sb, H * (DN + DR)).astype(_BF16)
    vv = _dot(cb, wv_ref[...]).reshape(sb, H, DV)
    vext = jnp.concatenate(
        [vv, jnp.ones((sb, H, 1), _F32), jnp.zeros((sb, H, DV2 - DV - 1), _F32)],
        axis=2)
    v_out[...] = vext.reshape(sb, H * DV2).astype(_BF16)


def _attn_kernel(qc_ref, kc_ref, v_ref, o_ref):
    iq = pl.program_id(1)
    qiota = jax.lax.broadcasted_iota(jnp.int32, (QB, KB), 0)
    kiota = jax.lax.broadcasted_iota(jnp.int32, (QB, KB), 1)
    qcs = [qc_ref[hh] for hh in range(HB)]

    def body(j, carry):
        out = []
        mask = j * KB + kiota <= iq * QB + qiota
        for hh in range(HB):
            m, l, acc = carry[hh]
            s = _dot_t(qcs[hh], kc_ref[hh, pl.ds(j * KB, KB), :])
            s = jnp.where(mask, s, -1e30)
            mn = jnp.maximum(m, jnp.max(s, axis=1, keepdims=True))
            p = jnp.exp(s - mn)
            corr = jnp.exp(m - mn)
            pv = _dot(p.astype(_BF16), v_ref[hh, pl.ds(j * KB, KB), :])
            l2 = l * corr + pv[:, DV:DV + 1]
            acc2 = acc * corr + pv[:, :DV]
            out.append((mn, l2, acc2))
        return tuple(out)

    init = tuple((jnp.full((QB, 1), -1e30, _F32),
                  jnp.zeros((QB, 1), _F32),
                  jnp.zeros((QB, DV), _F32)) for _ in range(HB))
    res = jax.lax.fori_loop(0, iq + 1, body, init)
    for hh in range(HB):
        m, l, acc = res[hh]
        o_ref[hh] = (acc / l).astype(_BF16)


def _proj_kernel(x_ref, a_ref, wo_ref, n2_ref, rw_ref, x1_ref, h2_ref,
                 lg_ref):
    x1 = x_ref[...] + _dot(a_ref[...], wo_ref[...])
    x1_ref[...] = x1
    h2 = x1 * jax.lax.rsqrt(jnp.mean(x1 * x1, axis=1, keepdims=True) + EPS)
    h2 = h2 * n2_ref[...]
    h2_ref[...] = h2.astype(_BF16)
    lg_ref[...] = _dot(h2, rw_ref[...])


def _router_kernel(lg_ref, gates_ref, aux_ref):
    lg = lg_ref[...]
    T = lg.shape[0]
    mx = jnp.max(lg, axis=1, keepdims=True)
    ex = jnp.exp(lg - mx)
    p = ex / jnp.sum(ex, axis=1, keepdims=True)
    iota = jax.lax.broadcasted_iota(jnp.int32, p.shape, 1)
    m1 = jnp.max(p, axis=1, keepdims=True)
    i1 = jnp.min(jnp.where(p >= m1, iota, E), axis=1, keepdims=True)
    oh1 = iota == i1
    pm = jnp.where(oh1, -1.0, p)
    m2 = jnp.max(pm, axis=1, keepdims=True)
    i2 = jnp.min(jnp.where(pm >= m2, iota, E), axis=1, keepdims=True)
    oh2 = iota == i2
    den = m1 + m2
    gates_ref[...] = (jnp.where(oh1, m1 / den, 0.0)
                      + jnp.where(oh2, m2 / den, 0.0))
    cnt = jnp.sum(oh1.astype(_F32) + oh2.astype(_F32), axis=0, keepdims=True)
    pmean = jnp.sum(p, axis=0, keepdims=True)
    f = cnt / (T * TOPK)
    pm_avg = pmean / T
    aux_ref[...] = ALPHA * E * jnp.sum(f * pm_avg, axis=1, keepdims=True)


def _moe_kernel(h2_ref, x1_ref, gates_ref, w1_ref, b1_ref, w2_ref, b2_ref,
                out_ref):
    e = pl.program_id(0)

    @pl.when(e == 0)
    def _():
        out_ref[...] = x1_ref[...]

    up = _dot(h2_ref[...], w1_ref[0]) + b1_ref[0]
    act = jax.nn.gelu(up)
    dn = _dot(act.astype(_BF16), w2_ref[0]) + b2_ref[0]
    gates = gates_ref[...]
    lane = jax.lax.broadcasted_iota(jnp.int32, gates.shape, 1)
    g = jnp.sum(jnp.where(lane == e, gates, 0.0), axis=1, keepdims=True)
    out_ref[...] += dn * g


def kernel(x, position_ids, norm1_w, Wq, Wdkv, kv_norm_w, Wukv, Wo, norm2_w,
           router_w, W1, b1, W2, b2):
    B, S, _ = x.shape
    x2d = x.reshape(S, D)

    inv = 1.0 / (THETA ** (jnp.arange(0, DR, 2, dtype=_F32) / DR))
    ang = position_ids.reshape(S).astype(_F32)[:, None] * inv
    cos = jnp.cos(ang)
    sin = jnp.sin(ang)
    cosq = jnp.tile(cos, (1, H))
    sinq = jnp.tile(sin, (1, H))

    Wq3 = Wq.reshape(D, H, DN + DR)
    wqn = Wq3[:, :, :DN].reshape(D, H * DN).astype(_BF16)
    wq1 = Wq3[:, :, DN::2].reshape(D, H * (DR // 2)).astype(_BF16)
    wq2 = Wq3[:, :, DN + 1::2].reshape(D, H * (DR // 2)).astype(_BF16)
    wdl = Wdkv[:, :L].astype(_BF16)
    wd1 = Wdkv[:, L::2].astype(_BF16)
    wd2 = Wdkv[:, L + 1::2].astype(_BF16)
    Wukv3 = Wukv.reshape(L, H, DN + DV)
    wkn = Wukv3[:, :, :DN].reshape(L, H * DN).astype(_BF16)
    wv = Wukv3[:, :, DN:].reshape(L, H * DV).astype(_BF16)
    wo = Wo.astype(_BF16)
    w1 = W1.astype(_BF16)
    w2 = W2.astype(_BF16)
    n1 = norm1_w.reshape(1, D)
    n2 = norm2_w.reshape(1, D)
    kvn = kv_norm_w.reshape(1, L)

    HR = H * (DR // 2)

    nsb = S // SB
    qc, kc, v = pl.pallas_call(
        _prep_kernel,
        grid=(nsb,),
        in_specs=[
            pl.BlockSpec((SB, D), lambda i: (i, 0)),
            pl.BlockSpec((SB, DR // 2), lambda i: (i, 0)),
            pl.BlockSpec((SB, DR // 2), lambda i: (i, 0)),
            pl.BlockSpec((SB, HR), lambda i: (i, 0)),
            pl.BlockSpec((SB, HR), lambda i: (i, 0)),
            pl.BlockSpec((1, D), lambda i: (0, 0)),
            pl.BlockSpec((D, H * DN), lambda i: (0, 0)),
            pl.BlockSpec((D, HR), lambda i: (0, 0)),
            pl.BlockSpec((D, HR), lambda i: (0, 0)),
            pl.BlockSpec((D, L), lambda i: (0, 0)),
            pl.BlockSpec((D, DR // 2), lambda i: (0, 0)),
            pl.BlockSpec((D, DR // 2), lambda i: (0, 0)),
            pl.BlockSpec((1, L), lambda i: (0, 0)),
            pl.BlockSpec((L, H * DN), lambda i: (0, 0)),
            pl.BlockSpec((L, H * DV), lambda i: (0, 0)),
        ],
        out_specs=[
            pl.BlockSpec((SB, H * (DN + DR)), lambda i: (i, 0)),
            pl.BlockSpec((SB, H * (DN + DR)), lambda i: (i, 0)),
            pl.BlockSpec((SB, H * DV2), lambda i: (i, 0)),
        ],
        out_shape=[
            jax.ShapeDtypeStruct((S, H * (DN + DR)), _BF16),
            jax.ShapeDtypeStruct((S, H * (DN + DR)), _BF16),
            jax.ShapeDtypeStruct((S, H * DV2), _BF16),
        ],
    )(x2d, cos, sin, cosq, sinq, n1, wqn, wq1, wq2, wdl, wd1, wd2, kvn,
      wkn, wv)

    DH = DN + DR
    qc3 = qc.reshape(S, H, DH).transpose(1, 0, 2)
    kc3 = kc.reshape(S, H, DH).transpose(1, 0, 2)
    v3 = v.reshape(S, H, DV2).transpose(1, 0, 2)

    nq = S // QB
    attn3 = pl.pallas_call(
        _attn_kernel,
        grid=(H // HB, nq),
        in_specs=[
            pl.BlockSpec((HB, QB, DH), lambda h, i: (h, i, 0)),
            pl.BlockSpec((HB, S, DH), lambda h, i: (h, 0, 0)),
            pl.BlockSpec((HB, S, DV2), lambda h, i: (h, 0, 0)),
        ],
        out_specs=pl.BlockSpec((HB, QB, DV), lambda h, i: (h, i, 0)),
        out_shape=jax.ShapeDtypeStruct((H, S, DV), _BF16),
    )(qc3, kc3, v3)
    attn = attn3.transpose(1, 0, 2).reshape(S, H * DV)

    x1, h2, logits = pl.pallas_call(
        _proj_kernel,
        grid=(nsb,),
        in_specs=[
            pl.BlockSpec((SB, D), lambda i: (i, 0)),
            pl.BlockSpec((SB, H * DV), lambda i: (i, 0)),
            pl.BlockSpec((H * DV, D), lambda i: (0, 0)),
            pl.BlockSpec((1, D), lambda i: (0, 0)),
            pl.BlockSpec((D, E), lambda i: (0, 0)),
        ],
        out_specs=[
            pl.BlockSpec((SB, D), lambda i: (i, 0)),
            pl.BlockSpec((SB, D), lambda i: (i, 0)),
            pl.BlockSpec((SB, E), lambda i: (i, 0)),
        ],
        out_shape=[
            jax.ShapeDtypeStruct((S, D), _F32),
            jax.ShapeDtypeStruct((S, D), _BF16),
            jax.ShapeDtypeStruct((S, E), _F32),
        ],
    )(x2d, attn, wo, n2, router_w)

    gates, aux = pl.pallas_call(
        _router_kernel,
        grid=(1,),
        in_specs=[pl.BlockSpec((S, E), lambda i: (0, 0))],
        out_specs=[
            pl.BlockSpec((S, E), lambda i: (0, 0)),
            pl.BlockSpec((1, 1), lambda i: (0, 0)),
        ],
        out_shape=[
            jax.ShapeDtypeStruct((S, E), _F32),
            jax.ShapeDtypeStruct((1, 1), _F32),
        ],
    )(logits)

    out = pl.pallas_call(
        _moe_kernel,
        grid=(E,),
        in_specs=[
            pl.BlockSpec((S, D), lambda e: (0, 0)),
            pl.BlockSpec((S, D), lambda e: (0, 0)),
            pl.BlockSpec((S, E), lambda e: (0, 0)),
            pl.BlockSpec((1, D, F), lambda e: (e, 0, 0)),
            pl.BlockSpec((1, 1, F), lambda e: (e, 0, 0)),
            pl.BlockSpec((1, F, D), lambda e: (e, 0, 0)),
            pl.BlockSpec((1, 1, D), lambda e: (e, 0, 0)),
        ],
        out_specs=pl.BlockSpec((S, D), lambda e: (0, 0)),
        out_shape=jax.ShapeDtypeStruct((S, D), _F32),
    )(h2, x1, gates, w1, b1.reshape(E, 1, F), w2, b2.reshape(E, 1, D))

    return (out.reshape(B, S, D), aux[0, 0])


# speedup vs baseline: 1.6831x; 1.3703x over previous
import jax
import jax.numpy as jnp
import numpy as np
from jax.experimental import pallas as pl
from jax.experimental.pallas import tpu as pltpu

D = 1024
H = 16
DN = 32
DR = 32
DV = 64
L = 256
E = 8
TOPK = 2
F = 512
THETA = 10000.0
ALPHA = 0.01
EPS = 1e-6
SCALE = 1.0 / np.sqrt(DN + DR)

SB = 512
QB = 256
KB = 256
DV2 = 128
HB = 4

_F32 = jnp.float32
_BF16 = jnp.bfloat16


def _dot(a, b):
    return jax.lax.dot_general(a, b, (((1,), (0,)), ((), ())),
                               preferred_element_type=_F32)


def _dot_t(a, b):
    return jax.lax.dot_general(a, b, (((1,), (1,)), ((), ())),
                               preferred_element_type=_F32)


def _prep_kernel(x_ref, cos_ref, sin_ref, cosq_ref, sinq_ref, n1_ref,
                 wqn_ref, wq1_ref, wq2_ref, wdl_ref, wd1_ref, wd2_ref,
                 kvn_ref, wkn_ref, wv_ref,
                 qc_out, kc_out, v_out):
    sb = x_ref.shape[0]
    x = x_ref[...]
    h = x * jax.lax.rsqrt(jnp.mean(x * x, axis=1, keepdims=True) + EPS)
    h = h * n1_ref[...]
    hb = h.astype(_BF16)

    qn = _dot(hb, wqn_ref[...])
    q1 = _dot(hb, wq1_ref[...])
    q2 = _dot(hb, wq2_ref[...])
    cq = cosq_ref[...]
    sq = sinq_ref[...]
    rq1 = q1 * cq - q2 * sq
    rq2 = q1 * sq + q2 * cq
    qc = jnp.concatenate(
        [(qn * SCALE).reshape(sb, H, DN),
         (rq1 * SCALE).reshape(sb, H, DR // 2),
         (rq2 * SCALE).reshape(sb, H, DR // 2)], axis=2)
    qc_out[...] = qc.reshape(sb, H * (DN + DR)).astype(_BF16)

    c = _dot(hb, wdl_ref[...])
    p1 = _dot(hb, wd1_ref[...])
    p2 = _dot(hb, wd2_ref[...])
    co = cos_ref[...]
    si = sin_ref[...]
    rk1 = p1 * co - p2 * si
    rk2 = p1 * si + p2 * co

    cn = c * jax.lax.rsqrt(jnp.mean(c * c, axis=1, keepdims=True) + EPS)
    cn = cn * kvn_ref[...]
    cb = cn.astype(_BF16)
    kn = _dot(cb, wkn_ref[...])
    kc = jnp.concatenate(
        [kn.reshape(sb, H, DN),
         jnp.broadcast_to(rk1[:, None, :], (sb, H, DR // 2)),
         jnp.broadcast_to(rk2[:, None, :], (sb, H, DR // 2))], axis=2)
    kc_out[...] = kc.reshape(sb, H * (DN + DR)).astype(_BF16)
    vv = _dot(cb, wv_ref[...]).reshape(sb, H, DV)
    vext = jnp.concatenate(
        [vv, jnp.ones((sb, H, 1), _F32), jnp.zeros((sb, H, DV2 - DV - 1), _F32)],
        axis=2)
    v_out[...] = vext.reshape(sb, H * DV2).astype(_BF16)


def _attn_kernel(qc_ref, kc_ref, v_ref, o_ref):
    iq = pl.program_id(1)
    qiota = jax.lax.broadcasted_iota(jnp.int32, (QB, KB), 0)
    kiota = jax.lax.broadcasted_iota(jnp.int32, (QB, KB), 1)
    DH = DN + DR
    qcs = [qc_ref[:, hh * DH:(hh + 1) * DH] for hh in range(HB)]

    def body(j, carry):
        out = []
        mask = j * KB + kiota <= iq * QB + qiota
        for hh in range(HB):
            m, l, acc = carry[hh]
            kcj = kc_ref[pl.ds(j * KB, KB), hh * DH:(hh + 1) * DH]
            s = _dot_t(qcs[hh], kcj)
            s = jnp.where(mask, s, -1e30)
            mn = jnp.maximum(m, jnp.max(s, axis=1, keepdims=True))
            p = jnp.exp(s - mn)
            corr = jnp.exp(m - mn)
            vj = v_ref[pl.ds(j * KB, KB), hh * DV2:(hh + 1) * DV2]
            pv = _dot(p.astype(_BF16), vj)
            l2 = l * corr + pv[:, DV:DV + 1]
            acc2 = acc * corr + pv[:, :DV]
            out.append((mn, l2, acc2))
        return tuple(out)

    init = tuple((jnp.full((QB, 1), -1e30, _F32),
                  jnp.zeros((QB, 1), _F32),
                  jnp.zeros((QB, DV), _F32)) for _ in range(HB))
    res = jax.lax.fori_loop(0, iq + 1, body, init)
    for hh in range(HB):
        m, l, acc = res[hh]
        o_ref[:, hh * DV:(hh + 1) * DV] = (acc / l).astype(_BF16)


def _proj_kernel(x_ref, a_ref, wo_ref, n2_ref, rw_ref, x1_ref, h2_ref,
                 lg_ref):
    x1 = x_ref[...] + _dot(a_ref[...], wo_ref[...])
    x1_ref[...] = x1
    h2 = x1 * jax.lax.rsqrt(jnp.mean(x1 * x1, axis=1, keepdims=True) + EPS)
    h2 = h2 * n2_ref[...]
    h2_ref[...] = h2.astype(_BF16)
    lg_ref[...] = _dot(h2, rw_ref[...])


def _router_kernel(lg_ref, gates_ref, aux_ref):
    lg = lg_ref[...]
    T = lg.shape[0]
    mx = jnp.max(lg, axis=1, keepdims=True)
    ex = jnp.exp(lg - mx)
    p = ex / jnp.sum(ex, axis=1, keepdims=True)
    iota = jax.lax.broadcasted_iota(jnp.int32, p.shape, 1)
    m1 = jnp.max(p, axis=1, keepdims=True)
    i1 = jnp.min(jnp.where(p >= m1, iota, E), axis=1, keepdims=True)
    oh1 = iota == i1
    pm = jnp.where(oh1, -1.0, p)
    m2 = jnp.max(pm, axis=1, keepdims=True)
    i2 = jnp.min(jnp.where(pm >= m2, iota, E), axis=1, keepdims=True)
    oh2 = iota == i2
    den = m1 + m2
    gates_ref[...] = (jnp.where(oh1, m1 / den, 0.0)
                      + jnp.where(oh2, m2 / den, 0.0))
    cnt = jnp.sum(oh1.astype(_F32) + oh2.astype(_F32), axis=0, keepdims=True)
    pmean = jnp.sum(p, axis=0, keepdims=True)
    f = cnt / (T * TOPK)
    pm_avg = pmean / T
    aux_ref[...] = ALPHA * E * jnp.sum(f * pm_avg, axis=1, keepdims=True)


def _moe_kernel(h2_ref, x1_ref, gates_ref, w1_ref, b1_ref, w2_ref, b2_ref,
                out_ref):
    e = pl.program_id(0)

    @pl.when(e == 0)
    def _():
        out_ref[...] = x1_ref[...]

    up = _dot(h2_ref[...], w1_ref[0]) + b1_ref[0]
    act = jax.nn.gelu(up)
    dn = _dot(act.astype(_BF16), w2_ref[0]) + b2_ref[0]
    gates = gates_ref[...]
    lane = jax.lax.broadcasted_iota(jnp.int32, gates.shape, 1)
    g = jnp.sum(jnp.where(lane == e, gates, 0.0), axis=1, keepdims=True)
    out_ref[...] += dn * g


def kernel(x, position_ids, norm1_w, Wq, Wdkv, kv_norm_w, Wukv, Wo, norm2_w,
           router_w, W1, b1, W2, b2):
    B, S, _ = x.shape
    x2d = x.reshape(S, D)

    inv = 1.0 / (THETA ** (jnp.arange(0, DR, 2, dtype=_F32) / DR))
    ang = position_ids.reshape(S).astype(_F32)[:, None] * inv
    cos = jnp.cos(ang)
    sin = jnp.sin(ang)
    cosq = jnp.tile(cos, (1, H))
    sinq = jnp.tile(sin, (1, H))

    Wq3 = Wq.reshape(D, H, DN + DR)
    wqn = Wq3[:, :, :DN].reshape(D, H * DN).astype(_BF16)
    wq1 = Wq3[:, :, DN::2].reshape(D, H * (DR // 2)).astype(_BF16)
    wq2 = Wq3[:, :, DN + 1::2].reshape(D, H * (DR // 2)).astype(_BF16)
    wdl = Wdkv[:, :L].astype(_BF16)
    wd1 = Wdkv[:, L::2].astype(_BF16)
    wd2 = Wdkv[:, L + 1::2].astype(_BF16)
    Wukv3 = Wukv.reshape(L, H, DN + DV)
    wkn = Wukv3[:, :, :DN].reshape(L, H * DN).astype(_BF16)
    wv = Wukv3[:, :, DN:].reshape(L, H * DV).astype(_BF16)
    wo = Wo.astype(_BF16)
    w1 = W1.astype(_BF16)
    w2 = W2.astype(_BF16)
    n1 = norm1_w.reshape(1, D)
    n2 = norm2_w.reshape(1, D)
    kvn = kv_norm_w.reshape(1, L)

    HR = H * (DR // 2)

    nsb = S // SB
    qc, kc, v = pl.pallas_call(
        _prep_kernel,
        grid=(nsb,),
        in_specs=[
            pl.BlockSpec((SB, D), lambda i: (i, 0)),
            pl.BlockSpec((SB, DR // 2), lambda i: (i, 0)),
            pl.BlockSpec((SB, DR // 2), lambda i: (i, 0)),
            pl.BlockSpec((SB, HR), lambda i: (i, 0)),
            pl.BlockSpec((SB, HR), lambda i: (i, 0)),
            pl.BlockSpec((1, D), lambda i: (0, 0)),
            pl.BlockSpec((D, H * DN), lambda i: (0, 0)),
            pl.BlockSpec((D, HR), lambda i: (0, 0)),
            pl.BlockSpec((D, HR), lambda i: (0, 0)),
            pl.BlockSpec((D, L), lambda i: (0, 0)),
            pl.BlockSpec((D, DR // 2), lambda i: (0, 0)),
            pl.BlockSpec((D, DR // 2), lambda i: (0, 0)),
            pl.BlockSpec((1, L), lambda i: (0, 0)),
            pl.BlockSpec((L, H * DN), lambda i: (0, 0)),
            pl.BlockSpec((L, H * DV), lambda i: (0, 0)),
        ],
        out_specs=[
            pl.BlockSpec((SB, H * (DN + DR)), lambda i: (i, 0)),
            pl.BlockSpec((SB, H * (DN + DR)), lambda i: (i, 0)),
            pl.BlockSpec((SB, H * DV2), lambda i: (i, 0)),
        ],
        out_shape=[
            jax.ShapeDtypeStruct((S, H * (DN + DR)), _BF16),
            jax.ShapeDtypeStruct((S, H * (DN + DR)), _BF16),
            jax.ShapeDtypeStruct((S, H * DV2), _BF16),
        ],
    )(x2d, cos, sin, cosq, sinq, n1, wqn, wq1, wq2, wdl, wd1, wd2, kvn,
      wkn, wv)

    DH = DN + DR
    nq = S // QB
    attn = pl.pallas_call(
        _attn_kernel,
        grid=(H // HB, nq),
        in_specs=[
            pl.BlockSpec((QB, HB * DH), lambda h, i: (i, h)),
            pl.BlockSpec((S, HB * DH), lambda h, i: (0, h)),
            pl.BlockSpec((S, HB * DV2), lambda h, i: (0, h)),
        ],
        out_specs=pl.BlockSpec((QB, HB * DV), lambda h, i: (i, h)),
        out_shape=jax.ShapeDtypeStruct((S, H * DV), _BF16),
    )(qc, kc, v)

    x1, h2, logits = pl.pallas_call(
        _proj_kernel,
        grid=(nsb,),
        in_specs=[
            pl.BlockSpec((SB, D), lambda i: (i, 0)),
            pl.BlockSpec((SB, H * DV), lambda i: (i, 0)),
            pl.BlockSpec((H * DV, D), lambda i: (0, 0)),
            pl.BlockSpec((1, D), lambda i: (0, 0)),
            pl.BlockSpec((D, E), lambda i: (0, 0)),
        ],
        out_specs=[
            pl.BlockSpec((SB, D), lambda i: (i, 0)),
            pl.BlockSpec((SB, D), lambda i: (i, 0)),
            pl.BlockSpec((SB, E), lambda i: (i, 0)),
        ],
        out_shape=[
            jax.ShapeDtypeStruct((S, D), _F32),
            jax.ShapeDtypeStruct((S, D), _BF16),
            jax.ShapeDtypeStruct((S, E), _F32),
        ],
    )(x2d, attn, wo, n2, router_w)

    gates, aux = pl.pallas_call(
        _router_kernel,
        grid=(1,),
        in_specs=[pl.BlockSpec((S, E), lambda i: (0, 0))],
        out_specs=[
            pl.BlockSpec((S, E), lambda i: (0, 0)),
            pl.BlockSpec((1, 1), lambda i: (0, 0)),
        ],
        out_shape=[
            jax.ShapeDtypeStruct((S, E), _F32),
            jax.ShapeDtypeStruct((1, 1), _F32),
        ],
    )(logits)

    out = pl.pallas_call(
        _moe_kernel,
        grid=(E,),
        in_specs=[
            pl.BlockSpec((S, D), lambda e: (0, 0)),
            pl.BlockSpec((S, D), lambda e: (0, 0)),
            pl.BlockSpec((S, E), lambda e: (0, 0)),
            pl.BlockSpec((1, D, F), lambda e: (e, 0, 0)),
            pl.BlockSpec((1, 1, F), lambda e: (e, 0, 0)),
            pl.BlockSpec((1, F, D), lambda e: (e, 0, 0)),
            pl.BlockSpec((1, 1, D), lambda e: (e, 0, 0)),
        ],
        out_specs=pl.BlockSpec((S, D), lambda e: (0, 0)),
        out_shape=jax.ShapeDtypeStruct((S, D), _F32),
    )(h2, x1, gates, w1, b1.reshape(E, 1, F), w2, b2.reshape(E, 1, D))

    return (out.reshape(B, S, D), aux[0, 0])


# speedup vs baseline: 1.6974x; 1.0085x over previous
import jax
import jax.numpy as jnp
import numpy as np
from jax.experimental import pallas as pl
from jax.experimental.pallas import tpu as pltpu

D = 1024
H = 16
DN = 32
DR = 32
DV = 64
L = 256
E = 8
TOPK = 2
F = 512
THETA = 10000.0
ALPHA = 0.01
EPS = 1e-6
SCALE = 1.0 / np.sqrt(DN + DR)

SB = 512
QB = 256
KB = 256
DV2 = 128
HB = 8

_F32 = jnp.float32
_BF16 = jnp.bfloat16


def _dot(a, b):
    return jax.lax.dot_general(a, b, (((1,), (0,)), ((), ())),
                               preferred_element_type=_F32)


def _dot_t(a, b):
    return jax.lax.dot_general(a, b, (((1,), (1,)), ((), ())),
                               preferred_element_type=_F32)


def _prep_kernel(x_ref, cos_ref, sin_ref, cosq_ref, sinq_ref, n1_ref,
                 wqn_ref, wq1_ref, wq2_ref, wdl_ref, wd1_ref, wd2_ref,
                 kvn_ref, wkn_ref, wv_ref,
                 qc_out, kc_out, v_out):
    sb = x_ref.shape[0]
    x = x_ref[...]
    h = x * jax.lax.rsqrt(jnp.mean(x * x, axis=1, keepdims=True) + EPS)
    h = h * n1_ref[...]
    hb = h.astype(_BF16)

    qn = _dot(hb, wqn_ref[...])
    q1 = _dot(hb, wq1_ref[...])
    q2 = _dot(hb, wq2_ref[...])
    cq = cosq_ref[...]
    sq = sinq_ref[...]
    rq1 = q1 * cq - q2 * sq
    rq2 = q1 * sq + q2 * cq
    qc = jnp.concatenate(
        [(qn * SCALE).reshape(sb, H, DN),
         (rq1 * SCALE).reshape(sb, H, DR // 2),
         (rq2 * SCALE).reshape(sb, H, DR // 2)], axis=2)
    qc_out[...] = qc.reshape(sb, H * (DN + DR)).astype(_BF16)

    c = _dot(hb, wdl_ref[...])
    p1 = _dot(hb, wd1_ref[...])
    p2 = _dot(hb, wd2_ref[...])
    co = cos_ref[...]
    si = sin_ref[...]
    rk1 = p1 * co - p2 * si
    rk2 = p1 * si + p2 * co

    cn = c * jax.lax.rsqrt(jnp.mean(c * c, axis=1, keepdims=True) + EPS)
    cn = cn * kvn_ref[...]
    cb = cn.astype(_BF16)
    kn = _dot(cb, wkn_ref[...])
    kc = jnp.concatenate(
        [kn.reshape(sb, H, DN),
         jnp.broadcast_to(rk1[:, None, :], (sb, H, DR // 2)),
         jnp.broadcast_to(rk2[:, None, :], (sb, H, DR // 2))], axis=2)
    kc_out[...] = kc.reshape(sb, H * (DN + DR)).astype(_BF16)
    vv = _dot(cb, wv_ref[...]).reshape(sb, H, DV)
    vext = jnp.concatenate(
        [vv, jnp.ones((sb, H, 1), _F32), jnp.zeros((sb, H, DV2 - DV - 1), _F32)],
        axis=2)
    v_out[...] = vext.reshape(sb, H * DV2).astype(_BF16)


def _attn_kernel(qc_ref, kc_ref, v_ref, o_ref):
    iq = pl.program_id(1)
    qiota = jax.lax.broadcasted_iota(jnp.int32, (QB, KB), 0)
    kiota = jax.lax.broadcasted_iota(jnp.int32, (QB, KB), 1)
    DH = DN + DR
    qcs = [qc_ref[:, hh * DH:(hh + 1) * DH] for hh in range(HB)]

    def body(j, carry):
        out = []
        mask = j * KB + kiota <= iq * QB + qiota
        for hh in range(HB):
            m, l, acc = carry[hh]
            kcj = kc_ref[pl.ds(j * KB, KB), hh * DH:(hh + 1) * DH]
            s = _dot_t(qcs[hh], kcj)
            s = jnp.where(mask, s, -1e30)
            mn = jnp.maximum(m, jnp.max(s, axis=1, keepdims=True))
            p = jnp.exp(s - mn)
            corr = jnp.exp(m - mn)
            vj = v_ref[pl.ds(j * KB, KB), hh * DV2:(hh + 1) * DV2]
            pv = _dot(p.astype(_BF16), vj)
            l2 = l * corr + pv[:, DV:DV + 1]
            acc2 = acc * corr + pv[:, :DV]
            out.append((mn, l2, acc2))
        return tuple(out)

    init = tuple((jnp.full((QB, 1), -1e30, _F32),
                  jnp.zeros((QB, 1), _F32),
                  jnp.zeros((QB, DV), _F32)) for _ in range(HB))
    res = jax.lax.fori_loop(0, iq + 1, body, init)
    for hh in range(HB):
        m, l, acc = res[hh]
        o_ref[:, hh * DV:(hh + 1) * DV] = (acc / l).astype(_BF16)


def _proj_kernel(x_ref, a_ref, wo_ref, n2_ref, rw_ref, x1_ref, h2_ref,
                 lg_ref):
    x1 = x_ref[...] + _dot(a_ref[...], wo_ref[...])
    x1_ref[...] = x1
    h2 = x1 * jax.lax.rsqrt(jnp.mean(x1 * x1, axis=1, keepdims=True) + EPS)
    h2 = h2 * n2_ref[...]
    h2_ref[...] = h2.astype(_BF16)
    lg_ref[...] = _dot(h2, rw_ref[...])


def _router_kernel(lg_ref, gates_ref, aux_ref):
    lg = lg_ref[...]
    T = lg.shape[0]
    mx = jnp.max(lg, axis=1, keepdims=True)
    ex = jnp.exp(lg - mx)
    p = ex / jnp.sum(ex, axis=1, keepdims=True)
    iota = jax.lax.broadcasted_iota(jnp.int32, p.shape, 1)
    m1 = jnp.max(p, axis=1, keepdims=True)
    i1 = jnp.min(jnp.where(p >= m1, iota, E), axis=1, keepdims=True)
    oh1 = iota == i1
    pm = jnp.where(oh1, -1.0, p)
    m2 = jnp.max(pm, axis=1, keepdims=True)
    i2 = jnp.min(jnp.where(pm >= m2, iota, E), axis=1, keepdims=True)
    oh2 = iota == i2
    den = m1 + m2
    gates_ref[...] = (jnp.where(oh1, m1 / den, 0.0)
                      + jnp.where(oh2, m2 / den, 0.0))
    cnt = jnp.sum(oh1.astype(_F32) + oh2.astype(_F32), axis=0, keepdims=True)
    pmean = jnp.sum(p, axis=0, keepdims=True)
    f = cnt / (T * TOPK)
    pm_avg = pmean / T
    aux_ref[...] = ALPHA * E * jnp.sum(f * pm_avg, axis=1, keepdims=True)


def _moe_kernel(h2_ref, x1_ref, gates_ref, w1_ref, b1_ref, w2_ref, b2_ref,
                out_ref):
    e = pl.program_id(0)

    @pl.when(e == 0)
    def _():
        out_ref[...] = x1_ref[...]

    up = _dot(h2_ref[...], w1_ref[0]) + b1_ref[0]
    act = jax.nn.gelu(up)
    dn = _dot(act.astype(_BF16), w2_ref[0]) + b2_ref[0]
    gates = gates_ref[...]
    lane = jax.lax.broadcasted_iota(jnp.int32, gates.shape, 1)
    g = jnp.sum(jnp.where(lane == e, gates, 0.0), axis=1, keepdims=True)
    out_ref[...] += dn * g


def kernel(x, position_ids, norm1_w, Wq, Wdkv, kv_norm_w, Wukv, Wo, norm2_w,
           router_w, W1, b1, W2, b2):
    B, S, _ = x.shape
    x2d = x.reshape(S, D)

    inv = 1.0 / (THETA ** (jnp.arange(0, DR, 2, dtype=_F32) / DR))
    ang = position_ids.reshape(S).astype(_F32)[:, None] * inv
    cos = jnp.cos(ang)
    sin = jnp.sin(ang)
    cosq = jnp.tile(cos, (1, H))
    sinq = jnp.tile(sin, (1, H))

    Wq3 = Wq.reshape(D, H, DN + DR)
    wqn = Wq3[:, :, :DN].reshape(D, H * DN).astype(_BF16)
    wq1 = Wq3[:, :, DN::2].reshape(D, H * (DR // 2)).astype(_BF16)
    wq2 = Wq3[:, :, DN + 1::2].reshape(D, H * (DR // 2)).astype(_BF16)
    wdl = Wdkv[:, :L].astype(_BF16)
    wd1 = Wdkv[:, L::2].astype(_BF16)
    wd2 = Wdkv[:, L + 1::2].astype(_BF16)
    Wukv3 = Wukv.reshape(L, H, DN + DV)
    wkn = Wukv3[:, :, :DN].reshape(L, H * DN).astype(_BF16)
    wv = Wukv3[:, :, DN:].reshape(L, H * DV).astype(_BF16)
    wo = Wo.astype(_BF16)
    w1 = W1.astype(_BF16)
    w2 = W2.astype(_BF16)
    n1 = norm1_w.reshape(1, D)
    n2 = norm2_w.reshape(1, D)
    kvn = kv_norm_w.reshape(1, L)

    HR = H * (DR // 2)

    nsb = S // SB
    qc, kc, v = pl.pallas_call(
        _prep_kernel,
        grid=(nsb,),
        in_specs=[
            pl.BlockSpec((SB, D), lambda i: (i, 0)),
            pl.BlockSpec((SB, DR // 2), lambda i: (i, 0)),
            pl.BlockSpec((SB, DR // 2), lambda i: (i, 0)),
            pl.BlockSpec((SB, HR), lambda i: (i, 0)),
            pl.BlockSpec((SB, HR), lambda i: (i, 0)),
            pl.BlockSpec((1, D), lambda i: (0, 0)),
            pl.BlockSpec((D, H * DN), lambda i: (0, 0)),
            pl.BlockSpec((D, HR), lambda i: (0, 0)),
            pl.BlockSpec((D, HR), lambda i: (0, 0)),
            pl.BlockSpec((D, L), lambda i: (0, 0)),
            pl.BlockSpec((D, DR // 2), lambda i: (0, 0)),
            pl.BlockSpec((D, DR // 2), lambda i: (0, 0)),
            pl.BlockSpec((1, L), lambda i: (0, 0)),
            pl.BlockSpec((L, H * DN), lambda i: (0, 0)),
            pl.BlockSpec((L, H * DV), lambda i: (0, 0)),
        ],
        out_specs=[
            pl.BlockSpec((SB, H * (DN + DR)), lambda i: (i, 0)),
            pl.BlockSpec((SB, H * (DN + DR)), lambda i: (i, 0)),
            pl.BlockSpec((SB, H * DV2), lambda i: (i, 0)),
        ],
        out_shape=[
            jax.ShapeDtypeStruct((S, H * (DN + DR)), _BF16),
            jax.ShapeDtypeStruct((S, H * (DN + DR)), _BF16),
            jax.ShapeDtypeStruct((S, H * DV2), _BF16),
        ],
    )(x2d, cos, sin, cosq, sinq, n1, wqn, wq1, wq2, wdl, wd1, wd2, kvn,
      wkn, wv)

    DH = DN + DR
    nq = S // QB
    attn = pl.pallas_call(
        _attn_kernel,
        grid=(H // HB, nq),
        in_specs=[
            pl.BlockSpec((QB, HB * DH), lambda h, i: (i, h)),
            pl.BlockSpec((S, HB * DH), lambda h, i: (0, h)),
            pl.BlockSpec((S, HB * DV2), lambda h, i: (0, h)),
        ],
        out_specs=pl.BlockSpec((QB, HB * DV), lambda h, i: (i, h)),
        out_shape=jax.ShapeDtypeStruct((S, H * DV), _BF16),
    )(qc, kc, v)

    x1, h2, logits = pl.pallas_call(
        _proj_kernel,
        grid=(nsb,),
        in_specs=[
            pl.BlockSpec((SB, D), lambda i: (i, 0)),
            pl.BlockSpec((SB, H * DV), lambda i: (i, 0)),
            pl.BlockSpec((H * DV, D), lambda i: (0, 0)),
            pl.BlockSpec((1, D), lambda i: (0, 0)),
            pl.BlockSpec((D, E), lambda i: (0, 0)),
        ],
        out_specs=[
            pl.BlockSpec((SB, D), lambda i: (i, 0)),
            pl.BlockSpec((SB, D), lambda i: (i, 0)),
            pl.BlockSpec((SB, E), lambda i: (i, 0)),
        ],
        out_shape=[
            jax.ShapeDtypeStruct((S, D), _F32),
            jax.ShapeDtypeStruct((S, D), _BF16),
            jax.ShapeDtypeStruct((S, E), _F32),
        ],
    )(x2d, attn, wo, n2, router_w)

    gates, aux = pl.pallas_call(
        _router_kernel,
        grid=(1,),
        in_specs=[pl.BlockSpec((S, E), lambda i: (0, 0))],
        out_specs=[
            pl.BlockSpec((S, E), lambda i: (0, 0)),
            pl.BlockSpec((1, 1), lambda i: (0, 0)),
        ],
        out_shape=[
            jax.ShapeDtypeStruct((S, E), _F32),
            jax.ShapeDtypeStruct((1, 1), _F32),
        ],
    )(logits)

    out = pl.pallas_call(
        _moe_kernel,
        grid=(E,),
        in_specs=[
            pl.BlockSpec((S, D), lambda e: (0, 0)),
            pl.BlockSpec((S, D), lambda e: (0, 0)),
            pl.BlockSpec((S, E), lambda e: (0, 0)),
            pl.BlockSpec((1, D, F), lambda e: (e, 0, 0)),
            pl.BlockSpec((1, 1, F), lambda e: (e, 0, 0)),
            pl.BlockSpec((1, F, D), lambda e: (e, 0, 0)),
            pl.BlockSpec((1, 1, D), lambda e: (e, 0, 0)),
        ],
        out_specs=pl.BlockSpec((S, D), lambda e: (0, 0)),
        out_shape=jax.ShapeDtypeStruct((S, D), _F32),
    )(h2, x1, gates, w1, b1.reshape(E, 1, F), w2, b2.reshape(E, 1, D))

    return (out.reshape(B, S, D), aux[0, 0])


# speedup vs baseline: 1.7566x; 1.0349x over previous
import jax
import jax.numpy as jnp
import numpy as np
from jax.experimental import pallas as pl
from jax.experimental.pallas import tpu as pltpu

D = 1024
H = 16
DN = 32
DR = 32
DV = 64
L = 256
E = 8
TOPK = 2
F = 512
THETA = 10000.0
ALPHA = 0.01
EPS = 1e-6
SCALE = 1.0 / np.sqrt(DN + DR)

SB = 512
QB = 256
KB = 512
DV2 = 128
HB = 8

_F32 = jnp.float32
_BF16 = jnp.bfloat16


def _dot(a, b):
    return jax.lax.dot_general(a, b, (((1,), (0,)), ((), ())),
                               preferred_element_type=_F32)


def _dot_t(a, b):
    return jax.lax.dot_general(a, b, (((1,), (1,)), ((), ())),
                               preferred_element_type=_F32)


def _prep_kernel(x_ref, cos_ref, sin_ref, cosq_ref, sinq_ref, n1_ref,
                 wqn_ref, wq1_ref, wq2_ref, wdl_ref, wd1_ref, wd2_ref,
                 kvn_ref, wkn_ref, wv_ref,
                 qc_out, kc_out, v_out):
    sb = x_ref.shape[0]
    x = x_ref[...]
    h = x * jax.lax.rsqrt(jnp.mean(x * x, axis=1, keepdims=True) + EPS)
    h = h * n1_ref[...]
    hb = h.astype(_BF16)

    qn = _dot(hb, wqn_ref[...])
    q1 = _dot(hb, wq1_ref[...])
    q2 = _dot(hb, wq2_ref[...])
    cq = cosq_ref[...]
    sq = sinq_ref[...]
    rq1 = q1 * cq - q2 * sq
    rq2 = q1 * sq + q2 * cq
    qc = jnp.concatenate(
        [(qn * SCALE).reshape(sb, H, DN),
         (rq1 * SCALE).reshape(sb, H, DR // 2),
         (rq2 * SCALE).reshape(sb, H, DR // 2)], axis=2)
    qc_out[...] = qc.reshape(sb, H * (DN + DR)).astype(_BF16)

    c = _dot(hb, wdl_ref[...])
    p1 = _dot(hb, wd1_ref[...])
    p2 = _dot(hb, wd2_ref[...])
    co = cos_ref[...]
    si = sin_ref[...]
    rk1 = p1 * co - p2 * si
    rk2 = p1 * si + p2 * co

    cn = c * jax.lax.rsqrt(jnp.mean(c * c, axis=1, keepdims=True) + EPS)
    cn = cn * kvn_ref[...]
    cb = cn.astype(_BF16)
    kn = _dot(cb, wkn_ref[...])
    kc = jnp.concatenate(
        [kn.reshape(sb, H, DN),
         jnp.broadcast_to(rk1[:, None, :], (sb, H, DR // 2)),
         jnp.broadcast_to(rk2[:, None, :], (sb, H, DR // 2))], axis=2)
    kc_out[...] = kc.reshape(sb, H * (DN + DR)).astype(_BF16)
    vv = _dot(cb, wv_ref[...]).reshape(sb, H, DV)
    vext = jnp.concatenate(
        [vv, jnp.ones((sb, H, 1), _F32), jnp.zeros((sb, H, DV2 - DV - 1), _F32)],
        axis=2)
    v_out[...] = vext.reshape(sb, H * DV2).astype(_BF16)


def _attn_kernel(qc_ref, kc_ref, v_ref, o_ref):
    iq = pl.program_id(1)
    qiota = jax.lax.broadcasted_iota(jnp.int32, (QB, KB), 0)
    kiota = jax.lax.broadcasted_iota(jnp.int32, (QB, KB), 1)
    DH = DN + DR
    qcs = [qc_ref[:, hh * DH:(hh + 1) * DH] for hh in range(HB)]

    def body(j, carry):
        out = []
        mask = j * KB + kiota <= iq * QB + qiota
        for hh in range(HB):
            m, l, acc = carry[hh]
            kcj = kc_ref[pl.ds(j * KB, KB), hh * DH:(hh + 1) * DH]
            s = _dot_t(qcs[hh], kcj)
            s = jnp.where(mask, s, -1e30)
            mn = jnp.maximum(m, jnp.max(s, axis=1, keepdims=True))
            p = jnp.exp(s - mn)
            corr = jnp.exp(m - mn)
            vj = v_ref[pl.ds(j * KB, KB), hh * DV2:(hh + 1) * DV2]
            pv = _dot(p.astype(_BF16), vj)
            l2 = l * corr + pv[:, DV:DV + 1]
            acc2 = acc * corr + pv[:, :DV]
            out.append((mn, l2, acc2))
        return tuple(out)

    init = tuple((jnp.full((QB, 1), -1e30, _F32),
                  jnp.zeros((QB, 1), _F32),
                  jnp.zeros((QB, DV), _F32)) for _ in range(HB))
    nkb = (iq * QB + QB + KB - 1) // KB
    res = jax.lax.fori_loop(0, nkb, body, init)
    for hh in range(HB):
        m, l, acc = res[hh]
        o_ref[:, hh * DV:(hh + 1) * DV] = (acc / l).astype(_BF16)


def _proj_kernel(x_ref, a_ref, wo_ref, n2_ref, rw_ref, x1_ref, h2_ref,
                 lg_ref):
    x1 = x_ref[...] + _dot(a_ref[...], wo_ref[...])
    x1_ref[...] = x1
    h2 = x1 * jax.lax.rsqrt(jnp.mean(x1 * x1, axis=1, keepdims=True) + EPS)
    h2 = h2 * n2_ref[...]
    h2_ref[...] = h2.astype(_BF16)
    lg_ref[...] = _dot(h2, rw_ref[...])


def _router_kernel(lg_ref, gates_ref, aux_ref):
    lg = lg_ref[...]
    T = lg.shape[0]
    mx = jnp.max(lg, axis=1, keepdims=True)
    ex = jnp.exp(lg - mx)
    p = ex / jnp.sum(ex, axis=1, keepdims=True)
    iota = jax.lax.broadcasted_iota(jnp.int32, p.shape, 1)
    m1 = jnp.max(p, axis=1, keepdims=True)
    i1 = jnp.min(jnp.where(p >= m1, iota, E), axis=1, keepdims=True)
    oh1 = iota == i1
    pm = jnp.where(oh1, -1.0, p)
    m2 = jnp.max(pm, axis=1, keepdims=True)
    i2 = jnp.min(jnp.where(pm >= m2, iota, E), axis=1, keepdims=True)
    oh2 = iota == i2
    den = m1 + m2
    gates_ref[...] = (jnp.where(oh1, m1 / den, 0.0)
                      + jnp.where(oh2, m2 / den, 0.0))
    cnt = jnp.sum(oh1.astype(_F32) + oh2.astype(_F32), axis=0, keepdims=True)
    pmean = jnp.sum(p, axis=0, keepdims=True)
    f = cnt / (T * TOPK)
    pm_avg = pmean / T
    aux_ref[...] = ALPHA * E * jnp.sum(f * pm_avg, axis=1, keepdims=True)


def _moe_kernel(h2_ref, x1_ref, gates_ref, w1_ref, b1_ref, w2_ref, b2_ref,
                out_ref):
    e = pl.program_id(0)

    @pl.when(e == 0)
    def _():
        out_ref[...] = x1_ref[...]

    up = _dot(h2_ref[...], w1_ref[0]) + b1_ref[0]
    act = jax.nn.gelu(up)
    dn = _dot(act.astype(_BF16), w2_ref[0]) + b2_ref[0]
    gates = gates_ref[...]
    lane = jax.lax.broadcasted_iota(jnp.int32, gates.shape, 1)
    g = jnp.sum(jnp.where(lane == e, gates, 0.0), axis=1, keepdims=True)
    out_ref[...] += dn * g


def kernel(x, position_ids, norm1_w, Wq, Wdkv, kv_norm_w, Wukv, Wo, norm2_w,
           router_w, W1, b1, W2, b2):
    B, S, _ = x.shape
    x2d = x.reshape(S, D)

    inv = 1.0 / (THETA ** (jnp.arange(0, DR, 2, dtype=_F32) / DR))
    ang = position_ids.reshape(S).astype(_F32)[:, None] * inv
    cos = jnp.cos(ang)
    sin = jnp.sin(ang)
    cosq = jnp.tile(cos, (1, H))
    sinq = jnp.tile(sin, (1, H))

    Wq3 = Wq.reshape(D, H, DN + DR)
    wqn = Wq3[:, :, :DN].reshape(D, H * DN).astype(_BF16)
    wq1 = Wq3[:, :, DN::2].reshape(D, H * (DR // 2)).astype(_BF16)
    wq2 = Wq3[:, :, DN + 1::2].reshape(D, H * (DR // 2)).astype(_BF16)
    wdl = Wdkv[:, :L].astype(_BF16)
    wd1 = Wdkv[:, L::2].astype(_BF16)
    wd2 = Wdkv[:, L + 1::2].astype(_BF16)
    Wukv3 = Wukv.reshape(L, H, DN + DV)
    wkn = Wukv3[:, :, :DN].reshape(L, H * DN).astype(_BF16)
    wv = Wukv3[:, :, DN:].reshape(L, H * DV).astype(_BF16)
    wo = Wo.astype(_BF16)
    w1 = W1.astype(_BF16)
    w2 = W2.astype(_BF16)
    n1 = norm1_w.reshape(1, D)
    n2 = norm2_w.reshape(1, D)
    kvn = kv_norm_w.reshape(1, L)

    HR = H * (DR // 2)

    nsb = S // SB
    qc, kc, v = pl.pallas_call(
        _prep_kernel,
        grid=(nsb,),
        in_specs=[
            pl.BlockSpec((SB, D), lambda i: (i, 0)),
            pl.BlockSpec((SB, DR // 2), lambda i: (i, 0)),
            pl.BlockSpec((SB, DR // 2), lambda i: (i, 0)),
            pl.BlockSpec((SB, HR), lambda i: (i, 0)),
            pl.BlockSpec((SB, HR), lambda i: (i, 0)),
            pl.BlockSpec((1, D), lambda i: (0, 0)),
            pl.BlockSpec((D, H * DN), lambda i: (0, 0)),
            pl.BlockSpec((D, HR), lambda i: (0, 0)),
            pl.BlockSpec((D, HR), lambda i: (0, 0)),
            pl.BlockSpec((D, L), lambda i: (0, 0)),
            pl.BlockSpec((D, DR // 2), lambda i: (0, 0)),
            pl.BlockSpec((D, DR // 2), lambda i: (0, 0)),
            pl.BlockSpec((1, L), lambda i: (0, 0)),
            pl.BlockSpec((L, H * DN), lambda i: (0, 0)),
            pl.BlockSpec((L, H * DV), lambda i: (0, 0)),
        ],
        out_specs=[
            pl.BlockSpec((SB, H * (DN + DR)), lambda i: (i, 0)),
            pl.BlockSpec((SB, H * (DN + DR)), lambda i: (i, 0)),
            pl.BlockSpec((SB, H * DV2), lambda i: (i, 0)),
        ],
        out_shape=[
            jax.ShapeDtypeStruct((S, H * (DN + DR)), _BF16),
            jax.ShapeDtypeStruct((S, H * (DN + DR)), _BF16),
            jax.ShapeDtypeStruct((S, H * DV2), _BF16),
        ],
    )(x2d, cos, sin, cosq, sinq, n1, wqn, wq1, wq2, wdl, wd1, wd2, kvn,
      wkn, wv)

    DH = DN + DR
    nq = S // QB
    attn = pl.pallas_call(
        _attn_kernel,
        grid=(H // HB, nq),
        in_specs=[
            pl.BlockSpec((QB, HB * DH), lambda h, i: (i, h)),
            pl.BlockSpec((S, HB * DH), lambda h, i: (0, h)),
            pl.BlockSpec((S, HB * DV2), lambda h, i: (0, h)),
        ],
        out_specs=pl.BlockSpec((QB, HB * DV), lambda h, i: (i, h)),
        out_shape=jax.ShapeDtypeStruct((S, H * DV), _BF16),
    )(qc, kc, v)

    x1, h2, logits = pl.pallas_call(
        _proj_kernel,
        grid=(nsb,),
        in_specs=[
            pl.BlockSpec((SB, D), lambda i: (i, 0)),
            pl.BlockSpec((SB, H * DV), lambda i: (i, 0)),
            pl.BlockSpec((H * DV, D), lambda i: (0, 0)),
            pl.BlockSpec((1, D), lambda i: (0, 0)),
            pl.BlockSpec((D, E), lambda i: (0, 0)),
        ],
        out_specs=[
            pl.BlockSpec((SB, D), lambda i: (i, 0)),
            pl.BlockSpec((SB, D), lambda i: (i, 0)),
            pl.BlockSpec((SB, E), lambda i: (i, 0)),
        ],
        out_shape=[
            jax.ShapeDtypeStruct((S, D), _F32),
            jax.ShapeDtypeStruct((S, D), _BF16),
            jax.ShapeDtypeStruct((S, E), _F32),
        ],
    )(x2d, attn, wo, n2, router_w)

    gates, aux = pl.pallas_call(
        _router_kernel,
        grid=(1,),
        in_specs=[pl.BlockSpec((S, E), lambda i: (0, 0))],
        out_specs=[
            pl.BlockSpec((S, E), lambda i: (0, 0)),
            pl.BlockSpec((1, 1), lambda i: (0, 0)),
        ],
        out_shape=[
            jax.ShapeDtypeStruct((S, E), _F32),
            jax.ShapeDtypeStruct((1, 1), _F32),
        ],
    )(logits)

    out = pl.pallas_call(
        _moe_kernel,
        grid=(E,),
        in_specs=[
            pl.BlockSpec((S, D), lambda e: (0, 0)),
            pl.BlockSpec((S, D), lambda e: (0, 0)),
            pl.BlockSpec((S, E), lambda e: (0, 0)),
            pl.BlockSpec((1, D, F), lambda e: (e, 0, 0)),
            pl.BlockSpec((1, 1, F), lambda e: (e, 0, 0)),
            pl.BlockSpec((1, F, D), lambda e: (e, 0, 0)),
            pl.BlockSpec((1, 1, D), lambda e: (e, 0, 0)),
        ],
        out_specs=pl.BlockSpec((S, D), lambda e: (0, 0)),
        out_shape=jax.ShapeDtypeStruct((S, D), _F32),
    )(h2, x1, gates, w1, b1.reshape(E, 1, F), w2, b2.reshape(E, 1, D))

    return (out.reshape(B, S, D), aux[0, 0])


# speedup vs baseline: 2.0262x; 1.1534x over previous
import jax
import jax.numpy as jnp
import numpy as np
from jax.experimental import pallas as pl
from jax.experimental.pallas import tpu as pltpu

D = 1024
H = 16
DN = 32
DR = 32
DV = 64
L = 256
E = 8
TOPK = 2
F = 512
THETA = 10000.0
ALPHA = 0.01
EPS = 1e-6
SCALE = 1.0 / np.sqrt(DN + DR)

SB = 512
QB = 512
KB = 512
DV2 = 128
HB = 8

_F32 = jnp.float32
_BF16 = jnp.bfloat16


def _dot(a, b):
    return jax.lax.dot_general(a, b, (((1,), (0,)), ((), ())),
                               preferred_element_type=_F32)


def _dot_t(a, b):
    return jax.lax.dot_general(a, b, (((1,), (1,)), ((), ())),
                               preferred_element_type=_F32)


def _prep_kernel(x_ref, cos_ref, sin_ref, cosq_ref, sinq_ref, n1_ref,
                 wqn_ref, wq1_ref, wq2_ref, wdl_ref, wd1_ref, wd2_ref,
                 kvn_ref, wkn_ref, wv_ref,
                 qc_out, kc_out, v_out):
    sb = x_ref.shape[0]
    x = x_ref[...]
    h = x * jax.lax.rsqrt(jnp.mean(x * x, axis=1, keepdims=True) + EPS)
    h = h * n1_ref[...]
    hb = h.astype(_BF16)

    qn = _dot(hb, wqn_ref[...])
    q1 = _dot(hb, wq1_ref[...])
    q2 = _dot(hb, wq2_ref[...])
    cq = cosq_ref[...]
    sq = sinq_ref[...]
    rq1 = q1 * cq - q2 * sq
    rq2 = q1 * sq + q2 * cq
    qc = jnp.concatenate(
        [(qn * SCALE).reshape(sb, H, DN),
         (rq1 * SCALE).reshape(sb, H, DR // 2),
         (rq2 * SCALE).reshape(sb, H, DR // 2)], axis=2)
    qc_out[...] = qc.reshape(sb, H * (DN + DR)).astype(_BF16)

    c = _dot(hb, wdl_ref[...])
    p1 = _dot(hb, wd1_ref[...])
    p2 = _dot(hb, wd2_ref[...])
    co = cos_ref[...]
    si = sin_ref[...]
    rk1 = p1 * co - p2 * si
    rk2 = p1 * si + p2 * co

    cn = c * jax.lax.rsqrt(jnp.mean(c * c, axis=1, keepdims=True) + EPS)
    cn = cn * kvn_ref[...]
    cb = cn.astype(_BF16)
    kn = _dot(cb, wkn_ref[...])
    kc = jnp.concatenate(
        [kn.reshape(sb, H, DN),
         jnp.broadcast_to(rk1[:, None, :], (sb, H, DR // 2)),
         jnp.broadcast_to(rk2[:, None, :], (sb, H, DR // 2))], axis=2)
    kc_out[...] = kc.reshape(sb, H * (DN + DR)).astype(_BF16)
    vv = _dot(cb, wv_ref[...]).reshape(sb, H, DV)
    vext = jnp.concatenate(
        [vv, jnp.ones((sb, H, 1), _F32), jnp.zeros((sb, H, DV2 - DV - 1), _F32)],
        axis=2)
    v_out[...] = vext.reshape(sb, H * DV2).astype(_BF16)


def _attn_kernel(qc_ref, kc_ref, v_ref, o_ref):
    iq = pl.program_id(1)
    qiota = jax.lax.broadcasted_iota(jnp.int32, (QB, KB), 0)
    kiota = jax.lax.broadcasted_iota(jnp.int32, (QB, KB), 1)
    DH = DN + DR
    qcs = [qc_ref[:, hh * DH:(hh + 1) * DH] for hh in range(HB)]

    def body(j, carry):
        out = []
        mask = j * KB + kiota <= iq * QB + qiota
        for hh in range(HB):
            m, l, acc = carry[hh]
            kcj = kc_ref[pl.ds(j * KB, KB), hh * DH:(hh + 1) * DH]
            s = _dot_t(qcs[hh], kcj)
            s = jnp.where(mask, s, -1e30)
            mn = jnp.maximum(m, jnp.max(s, axis=1, keepdims=True))
            p = jnp.exp(s - mn)
            corr = jnp.exp(m - mn)
            vj = v_ref[pl.ds(j * KB, KB), hh * DV2:(hh + 1) * DV2]
            pv = _dot(p.astype(_BF16), vj)
            l2 = l * corr + pv[:, DV:DV + 1]
            acc2 = acc * corr + pv[:, :DV]
            out.append((mn, l2, acc2))
        return tuple(out)

    init = tuple((jnp.full((QB, 1), -1e30, _F32),
                  jnp.zeros((QB, 1), _F32),
                  jnp.zeros((QB, DV), _F32)) for _ in range(HB))
    nkb = (iq * QB + QB + KB - 1) // KB
    res = jax.lax.fori_loop(0, nkb, body, init)
    for hh in range(HB):
        m, l, acc = res[hh]
        o_ref[:, hh * DV:(hh + 1) * DV] = (acc / l).astype(_BF16)


def _proj_kernel(x_ref, a_ref, wo_ref, n2_ref, rw_ref, x1_ref, h2_ref,
                 lg_ref):
    x1 = x_ref[...] + _dot(a_ref[...], wo_ref[...])
    x1_ref[...] = x1
    h2 = x1 * jax.lax.rsqrt(jnp.mean(x1 * x1, axis=1, keepdims=True) + EPS)
    h2 = h2 * n2_ref[...]
    h2_ref[...] = h2.astype(_BF16)
    lg_ref[...] = _dot(h2, rw_ref[...])


def _router_kernel(lg_ref, gates_ref, aux_ref):
    lg = lg_ref[...]
    T = lg.shape[0]
    mx = jnp.max(lg, axis=1, keepdims=True)
    ex = jnp.exp(lg - mx)
    p = ex / jnp.sum(ex, axis=1, keepdims=True)
    iota = jax.lax.broadcasted_iota(jnp.int32, p.shape, 1)
    m1 = jnp.max(p, axis=1, keepdims=True)
    i1 = jnp.min(jnp.where(p >= m1, iota, E), axis=1, keepdims=True)
    oh1 = iota == i1
    pm = jnp.where(oh1, -1.0, p)
    m2 = jnp.max(pm, axis=1, keepdims=True)
    i2 = jnp.min(jnp.where(pm >= m2, iota, E), axis=1, keepdims=True)
    oh2 = iota == i2
    den = m1 + m2
    gates_ref[...] = (jnp.where(oh1, m1 / den, 0.0)
                      + jnp.where(oh2, m2 / den, 0.0))
    cnt = jnp.sum(oh1.astype(_F32) + oh2.astype(_F32), axis=0, keepdims=True)
    pmean = jnp.sum(p, axis=0, keepdims=True)
    f = cnt / (T * TOPK)
    pm_avg = pmean / T
    aux_ref[...] = ALPHA * E * jnp.sum(f * pm_avg, axis=1, keepdims=True)


def _moe_kernel(h2_ref, x1_ref, gates_ref, w1_ref, b1_ref, w2_ref, b2_ref,
                out_ref):
    e = pl.program_id(0)

    @pl.when(e == 0)
    def _():
        out_ref[...] = x1_ref[...]

    up = _dot(h2_ref[...], w1_ref[0]) + b1_ref[0]
    act = jax.nn.gelu(up)
    dn = _dot(act.astype(_BF16), w2_ref[0]) + b2_ref[0]
    gates = gates_ref[...]
    lane = jax.lax.broadcasted_iota(jnp.int32, gates.shape, 1)
    g = jnp.sum(jnp.where(lane == e, gates, 0.0), axis=1, keepdims=True)
    out_ref[...] += dn * g


def kernel(x, position_ids, norm1_w, Wq, Wdkv, kv_norm_w, Wukv, Wo, norm2_w,
           router_w, W1, b1, W2, b2):
    B, S, _ = x.shape
    x2d = x.reshape(S, D)

    inv = 1.0 / (THETA ** (jnp.arange(0, DR, 2, dtype=_F32) / DR))
    ang = position_ids.reshape(S).astype(_F32)[:, None] * inv
    cos = jnp.cos(ang)
    sin = jnp.sin(ang)
    cosq = jnp.tile(cos, (1, H))
    sinq = jnp.tile(sin, (1, H))

    Wq3 = Wq.reshape(D, H, DN + DR)
    wqn = Wq3[:, :, :DN].reshape(D, H * DN).astype(_BF16)
    wq1 = Wq3[:, :, DN::2].reshape(D, H * (DR // 2)).astype(_BF16)
    wq2 = Wq3[:, :, DN + 1::2].reshape(D, H * (DR // 2)).astype(_BF16)
    wdl = Wdkv[:, :L].astype(_BF16)
    wd1 = Wdkv[:, L::2].astype(_BF16)
    wd2 = Wdkv[:, L + 1::2].astype(_BF16)
    Wukv3 = Wukv.reshape(L, H, DN + DV)
    wkn = Wukv3[:, :, :DN].reshape(L, H * DN).astype(_BF16)
    wv = Wukv3[:, :, DN:].reshape(L, H * DV).astype(_BF16)
    wo = Wo.astype(_BF16)
    w1 = W1.astype(_BF16)
    w2 = W2.astype(_BF16)
    n1 = norm1_w.reshape(1, D)
    n2 = norm2_w.reshape(1, D)
    kvn = kv_norm_w.reshape(1, L)

    HR = H * (DR // 2)

    nsb = S // SB
    qc, kc, v = pl.pallas_call(
        _prep_kernel,
        grid=(nsb,),
        in_specs=[
            pl.BlockSpec((SB, D), lambda i: (i, 0)),
            pl.BlockSpec((SB, DR // 2), lambda i: (i, 0)),
            pl.BlockSpec((SB, DR // 2), lambda i: (i, 0)),
            pl.BlockSpec((SB, HR), lambda i: (i, 0)),
            pl.BlockSpec((SB, HR), lambda i: (i, 0)),
            pl.BlockSpec((1, D), lambda i: (0, 0)),
            pl.BlockSpec((D, H * DN), lambda i: (0, 0)),
            pl.BlockSpec((D, HR), lambda i: (0, 0)),
            pl.BlockSpec((D, HR), lambda i: (0, 0)),
            pl.BlockSpec((D, L), lambda i: (0, 0)),
            pl.BlockSpec((D, DR // 2), lambda i: (0, 0)),
            pl.BlockSpec((D, DR // 2), lambda i: (0, 0)),
            pl.BlockSpec((1, L), lambda i: (0, 0)),
            pl.BlockSpec((L, H * DN), lambda i: (0, 0)),
            pl.BlockSpec((L, H * DV), lambda i: (0, 0)),
        ],
        out_specs=[
            pl.BlockSpec((SB, H * (DN + DR)), lambda i: (i, 0)),
            pl.BlockSpec((SB, H * (DN + DR)), lambda i: (i, 0)),
            pl.BlockSpec((SB, H * DV2), lambda i: (i, 0)),
        ],
        out_shape=[
            jax.ShapeDtypeStruct((S, H * (DN + DR)), _BF16),
            jax.ShapeDtypeStruct((S, H * (DN + DR)), _BF16),
            jax.ShapeDtypeStruct((S, H * DV2), _BF16),
        ],
    )(x2d, cos, sin, cosq, sinq, n1, wqn, wq1, wq2, wdl, wd1, wd2, kvn,
      wkn, wv)

    DH = DN + DR
    nq = S // QB
    attn = pl.pallas_call(
        _attn_kernel,
        grid=(H // HB, nq),
        in_specs=[
            pl.BlockSpec((QB, HB * DH), lambda h, i: (i, h)),
            pl.BlockSpec((S, HB * DH), lambda h, i: (0, h)),
            pl.BlockSpec((S, HB * DV2), lambda h, i: (0, h)),
        ],
        out_specs=pl.BlockSpec((QB, HB * DV), lambda h, i: (i, h)),
        out_shape=jax.ShapeDtypeStruct((S, H * DV), _BF16),
    )(qc, kc, v)

    x1, h2, logits = pl.pallas_call(
        _proj_kernel,
        grid=(nsb,),
        in_specs=[
            pl.BlockSpec((SB, D), lambda i: (i, 0)),
            pl.BlockSpec((SB, H * DV), lambda i: (i, 0)),
            pl.BlockSpec((H * DV, D), lambda i: (0, 0)),
            pl.BlockSpec((1, D), lambda i: (0, 0)),
            pl.BlockSpec((D, E), lambda i: (0, 0)),
        ],
        out_specs=[
            pl.BlockSpec((SB, D), lambda i: (i, 0)),
            pl.BlockSpec((SB, D), lambda i: (i, 0)),
            pl.BlockSpec((SB, E), lambda i: (i, 0)),
        ],
        out_shape=[
            jax.ShapeDtypeStruct((S, D), _F32),
            jax.ShapeDtypeStruct((S, D), _BF16),
            jax.ShapeDtypeStruct((S, E), _F32),
        ],
    )(x2d, attn, wo, n2, router_w)

    gates, aux = pl.pallas_call(
        _router_kernel,
        grid=(1,),
        in_specs=[pl.BlockSpec((S, E), lambda i: (0, 0))],
        out_specs=[
            pl.BlockSpec((S, E), lambda i: (0, 0)),
            pl.BlockSpec((1, 1), lambda i: (0, 0)),
        ],
        out_shape=[
            jax.ShapeDtypeStruct((S, E), _F32),
            jax.ShapeDtypeStruct((1, 1), _F32),
        ],
    )(logits)

    out = pl.pallas_call(
        _moe_kernel,
        grid=(E,),
        in_specs=[
            pl.BlockSpec((S, D), lambda e: (0, 0)),
            pl.BlockSpec((S, D), lambda e: (0, 0)),
            pl.BlockSpec((S, E), lambda e: (0, 0)),
            pl.BlockSpec((1, D, F), lambda e: (e, 0, 0)),
            pl.BlockSpec((1, 1, F), lambda e: (e, 0, 0)),
            pl.BlockSpec((1, F, D), lambda e: (e, 0, 0)),
            pl.BlockSpec((1, 1, D), lambda e: (e, 0, 0)),
        ],
        out_specs=pl.BlockSpec((S, D), lambda e: (0, 0)),
        out_shape=jax.ShapeDtypeStruct((S, D), _F32),
    )(h2, x1, gates, w1, b1.reshape(E, 1, F), w2, b2.reshape(E, 1, D))

    return (out.reshape(B, S, D), aux[0, 0])
